# baseline (speedup 1.0000x reference)
import jax
import jax.numpy as jnp
from jax.experimental import pallas as pl
from jax.experimental.pallas import tpu as pltpu

_L = 50
_B = 4096
_D = 128


def _body(idx_ref, cc_ref, x_ref, buf_ref, out_ref, nbuf_ref,
          scratch_ref, row_sem, out_sem):
    idx = idx_ref[0]
    cc = cc_ref[0]

    @pl.when(cc >= _L)
    def _gather_out():
        pltpu.make_async_copy(buf_ref.at[idx], out_ref, out_sem).start()

    @pl.when(cc < _L)
    def _zero_out():
        scratch_ref[...] = jnp.zeros_like(scratch_ref)
        pltpu.make_async_copy(scratch_ref, out_ref, out_sem).start()

    for i in range(_L):
        @pl.when(idx != i)
        def _copy_row(i=i):
            pltpu.make_async_copy(buf_ref.at[i], nbuf_ref.at[i],
                                  row_sem).start()

        @pl.when(idx == i)
        def _write_slot(i=i):
            pltpu.make_async_copy(x_ref, nbuf_ref.at[i], row_sem).start()

    pltpu.make_async_copy(buf_ref.at[0], out_ref, out_sem).wait()
    for i in range(_L):
        pltpu.make_async_copy(buf_ref.at[i], nbuf_ref.at[i], row_sem).wait()


def kernel(x, buffer, index, call_count):
    idx = jnp.asarray(index, jnp.int32).reshape(1)
    cc = jnp.asarray(call_count, jnp.int32).reshape(1)
    output, new_buffer = pl.pallas_call(
        _body,
        in_specs=[
            pl.BlockSpec(memory_space=pltpu.MemorySpace.SMEM),
            pl.BlockSpec(memory_space=pltpu.MemorySpace.SMEM),
            pl.BlockSpec(memory_space=pl.ANY),
            pl.BlockSpec(memory_space=pl.ANY),
        ],
        out_specs=[
            pl.BlockSpec(memory_space=pl.ANY),
            pl.BlockSpec(memory_space=pl.ANY),
        ],
        out_shape=(
            jax.ShapeDtypeStruct((_B, _D), x.dtype),
            jax.ShapeDtypeStruct((_L, _B, _D), buffer.dtype),
        ),
        scratch_shapes=[
            pltpu.VMEM((_B, _D), jnp.float32),
            pltpu.SemaphoreType.DMA,
            pltpu.SemaphoreType.DMA,
        ],
    )(idx, cc, x, buffer)
    return output, new_buffer


# speedup vs baseline: 46.5461x; 46.5461x over previous
import jax
import jax.numpy as jnp
from jax.experimental import pallas as pl
from jax.experimental.pallas import tpu as pltpu

_L = 50
_B = 4096
_D = 128
_R = 2


def _body(idx_ref, cc_ref, x_ref, buf_ref, out_ref, nbuf_ref):
    i = pl.program_id(0)
    r = idx_ref[0] - _R * i
    nbuf_ref[...] = buf_ref[...]
    hit = jnp.logical_and(r >= 0, r < _R)

    @pl.when(hit)
    def _write_slot():
        nbuf_ref[r] = x_ref[...]
        out_ref[...] = jnp.where(cc_ref[0] >= _L, buf_ref[r],
                                 jnp.zeros_like(x_ref))


def kernel(x, buffer, index, call_count):
    idx = jnp.asarray(index, jnp.int32).reshape(1)
    cc = jnp.asarray(call_count, jnp.int32).reshape(1)
    grid_spec = pltpu.PrefetchScalarGridSpec(
        num_scalar_prefetch=2,
        grid=(_L // _R,),
        in_specs=[
            pl.BlockSpec((_B, _D), lambda i, *_: (0, 0)),
            pl.BlockSpec((_R, _B, _D), lambda i, *_: (i, 0, 0)),
        ],
        out_specs=[
            pl.BlockSpec((_B, _D), lambda i, *_: (0, 0)),
            pl.BlockSpec((_R, _B, _D), lambda i, *_: (i, 0, 0)),
        ],
    )
    output, new_buffer = pl.pallas_call(
        _body,
        grid_spec=grid_spec,
        out_shape=(
            jax.ShapeDtypeStruct((_B, _D), x.dtype),
            jax.ShapeDtypeStruct((_L, _B, _D), buffer.dtype),
        ),
    )(idx, cc, x, buffer)
    return output, new_buffer


# speedup vs baseline: 47.8320x; 1.0276x over previous
import jax
import jax.numpy as jnp
from jax.experimental import pallas as pl
from jax.experimental.pallas import tpu as pltpu

_L = 50
_B = 4096
_D = 128
_R = 5


def _body(idx_ref, cc_ref, x_ref, buf_ref, out_ref, nbuf_ref):
    i = pl.program_id(0)
    r = idx_ref[0] - _R * i
    nbuf_ref[...] = buf_ref[...]
    hit = jnp.logical_and(r >= 0, r < _R)

    @pl.when(hit)
    def _write_slot():
        nbuf_ref[r] = x_ref[...]
        out_ref[...] = jnp.where(cc_ref[0] >= _L, buf_ref[r],
                                 jnp.zeros_like(x_ref))


def kernel(x, buffer, index, call_count):
    idx = jnp.asarray(index, jnp.int32).reshape(1)
    cc = jnp.asarray(call_count, jnp.int32).reshape(1)
    grid_spec = pltpu.PrefetchScalarGridSpec(
        num_scalar_prefetch=2,
        grid=(_L // _R,),
        in_specs=[
            pl.BlockSpec((_B, _D), lambda i, *_: (0, 0)),
            pl.BlockSpec((_R, _B, _D), lambda i, *_: (i, 0, 0)),
        ],
        out_specs=[
            pl.BlockSpec((_B, _D), lambda i, *_: (0, 0)),
            pl.BlockSpec((_R, _B, _D), lambda i, *_: (i, 0, 0)),
        ],
    )
    output, new_buffer = pl.pallas_call(
        _body,
        grid_spec=grid_spec,
        out_shape=(
            jax.ShapeDtypeStruct((_B, _D), x.dtype),
            jax.ShapeDtypeStruct((_L, _B, _D), buffer.dtype),
        ),
    )(idx, cc, x, buffer)
    return output, new_buffer


# speedup vs baseline: 48.5205x; 1.0144x over previous
import jax
import jax.numpy as jnp
from jax.experimental import pallas as pl
from jax.experimental.pallas import tpu as pltpu

_L = 50
_B = 4096
_D = 128
_R = 6


def _body(idx_ref, cc_ref, x_ref, buf_ref, out_ref, nbuf_ref):
    i = pl.program_id(0)
    r = idx_ref[0] - _R * i
    nbuf_ref[...] = buf_ref[...]
    hit = jnp.logical_and(r >= 0, r < _R)

    @pl.when(hit)
    def _write_slot():
        nbuf_ref[r] = x_ref[...]
        out_ref[...] = jnp.where(cc_ref[0] >= _L, buf_ref[r],
                                 jnp.zeros_like(x_ref))


def kernel(x, buffer, index, call_count):
    idx = jnp.asarray(index, jnp.int32).reshape(1)
    cc = jnp.asarray(call_count, jnp.int32).reshape(1)
    grid_spec = pltpu.PrefetchScalarGridSpec(
        num_scalar_prefetch=2,
        grid=(pl.cdiv(_L, _R),),
        in_specs=[
            pl.BlockSpec((_B, _D), lambda i, *_: (0, 0)),
            pl.BlockSpec((_R, _B, _D), lambda i, *_: (i, 0, 0)),
        ],
        out_specs=[
            pl.BlockSpec((_B, _D), lambda i, *_: (0, 0)),
            pl.BlockSpec((_R, _B, _D), lambda i, *_: (i, 0, 0)),
        ],
    )
    output, new_buffer = pl.pallas_call(
        _body,
        grid_spec=grid_spec,
        out_shape=(
            jax.ShapeDtypeStruct((_B, _D), x.dtype),
            jax.ShapeDtypeStruct((_L, _B, _D), buffer.dtype),
        ),
    )(idx, cc, x, buffer)
    return output, new_buffer
